# baseline (speedup 1.0000x reference)
import functools

import jax
import jax.numpy as jnp
from jax import lax
from jax.experimental import pallas as pl
from jax.experimental.pallas import tpu as pltpu
from jax.experimental.pallas import tpu_sc as plsc

H = 1024
I = 704
E = 64
S = 2048
P = S * 2
TM = 256
NT = P // TM + E

NC = 2
NS = 16
NW = NC * NS
TOK_W = S // NW
CH = 16


def _router_plan_body(x_ref, gw_ref, d0_ref, d1_ref, w0_ref, w1_ref, meta_ref):
    x = x_ref[...]
    gw = gw_ref[...]
    logits = lax.dot_general(
        x.astype(jnp.bfloat16), gw.astype(jnp.bfloat16),
        (((1,), (1,)), ((), ())),
        preferred_element_type=jnp.float32)

    col = lax.broadcasted_iota(jnp.int32, (S, E), 1)
    m1 = jnp.max(logits, axis=1, keepdims=True)
    a1 = jnp.min(jnp.where(logits == m1, col, E), axis=1, keepdims=True)
    logits2 = jnp.where(col == a1, -jnp.inf, logits)
    m2 = jnp.max(logits2, axis=1, keepdims=True)
    a2 = jnp.min(jnp.where(logits2 == m2, col, E), axis=1, keepdims=True)

    t = jnp.exp(m2 - m1)
    w0_ref[...] = 1.0 / (1.0 + t)
    w1_ref[...] = t / (1.0 + t)

    oh0 = (col == a1).astype(jnp.float32)
    oh1 = (col == a2).astype(jnp.float32)
    csum = oh0 + oh1
    k = 1
    while k < S:
        pad = jnp.zeros((k, E), jnp.float32)
        csum = csum + jnp.concatenate([pad, csum[:-k, :]], axis=0)
        k *= 2
    cnt = csum[S - 1:S, :]

    er = lax.broadcasted_iota(jnp.int32, (E, E), 0)
    ec = lax.broadcasted_iota(jnp.int32, (E, E), 1)
    off = jnp.sum(jnp.where(ec < er, jnp.broadcast_to(cnt, (E, E)), 0.0),
                  axis=1).reshape(1, E)

    rank0 = jnp.sum(oh0 * csum, axis=1, keepdims=True) - 1.0
    rank1 = jnp.sum(oh1 * csum, axis=1, keepdims=True) - 1.0
    offb = jnp.broadcast_to(off, (S, E))
    d0 = jnp.sum(oh0 * offb, axis=1, keepdims=True) + rank0
    d1 = jnp.sum(oh1 * offb, axis=1, keepdims=True) + rank1
    d0_ref[...] = d0.astype(jnp.int32)
    d1_ref[...] = d1.astype(jnp.int32)

    tlo = jnp.floor(off / TM)
    thi = jnp.floor((off + cnt - 1.0) / TM)
    ntile = jnp.where(cnt > 0.0, thi - tlo + 1.0, 0.0)
    cinc = jnp.sum(jnp.where(ec <= er, jnp.broadcast_to(ntile, (E, E)), 0.0),
                   axis=1).reshape(1, E)
    cexc = cinc - ntile
    total = jnp.sum(ntile)

    s_idx = lax.broadcasted_iota(jnp.int32, (NT, 1), 0).astype(jnp.float32)
    s_c = jnp.minimum(s_idx, total - 1.0)
    cincb = jnp.broadcast_to(cinc, (NT, E))
    e_of_s = jnp.sum((s_c >= cincb).astype(jnp.float32), axis=1,
                     keepdims=True)
    colN = lax.broadcasted_iota(jnp.int32, (NT, E), 1).astype(jnp.float32)
    ohs = (colN == e_of_s).astype(jnp.float32)
    cexc_s = jnp.sum(ohs * jnp.broadcast_to(cexc, (NT, E)), axis=1,
                     keepdims=True)
    tlo_s = jnp.sum(ohs * jnp.broadcast_to(tlo, (NT, E)), axis=1,
                    keepdims=True)
    off_s = jnp.sum(ohs * jnp.broadcast_to(off, (NT, E)), axis=1,
                    keepdims=True)
    end_s = jnp.sum(ohs * jnp.broadcast_to(off + cnt, (NT, E)), axis=1,
                    keepdims=True)
    tile_s = tlo_s + (s_c - cexc_s)
    row_lo = jnp.maximum(off_s, tile_s * TM)
    row_hi = jnp.minimum(end_s, (tile_s + 1.0) * TM)
    valid = s_idx < total
    row_lo = jnp.where(valid, row_lo, 0.0)
    row_hi = jnp.where(valid, row_hi, 0.0)
    meta = jnp.concatenate([e_of_s, tile_s, row_lo, row_hi], axis=1)
    meta_ref[...] = meta.astype(jnp.int32)


def _router_plan(x2, gate_w):
    return pl.pallas_call(
        _router_plan_body,
        out_shape=(
            jax.ShapeDtypeStruct((S, 1), jnp.int32),
            jax.ShapeDtypeStruct((S, 1), jnp.int32),
            jax.ShapeDtypeStruct((S, 1), jnp.float32),
            jax.ShapeDtypeStruct((S, 1), jnp.float32),
            jax.ShapeDtypeStruct((NT, 4), jnp.int32),
        ),
    )(x2, gate_w)


def _ffn_body(meta_ref, d0_ref, d1_ref, rw0_ref, rw1_ref, xs_ref,
              w1_ref, up_ref, w2_ref, out_ref):
    i = pl.program_id(0)
    row_lo = meta_ref[i, 2]
    row_hi = meta_ref[i, 3]
    base = meta_ref[i, 1] * TM

    @pl.when(i == 0)
    def _init():
        out_ref[...] = jnp.zeros((S, H), jnp.float32)

    xb = xs_ref[...].astype(jnp.bfloat16)
    w1b = w1_ref[...].astype(jnp.bfloat16)
    upb = up_ref[...].astype(jnp.bfloat16)
    w2b = w2_ref[...].astype(jnp.bfloat16)
    nt = (((1,), (1,)), ((), ()))
    g = lax.dot_general(xb, w1b, nt, preferred_element_type=jnp.float32)
    u = lax.dot_general(xb, upb, nt, preferred_element_type=jnp.float32)
    h = (g / (1.0 + jnp.exp(-g))) * u
    y = lax.dot_general(h.astype(jnp.bfloat16), w2b, nt,
                        preferred_element_type=jnp.float32)

    riota = base + lax.broadcasted_iota(jnp.int32, (S, TM), 1)
    inr = (riota >= row_lo) & (riota < row_hi)
    sel0 = (d0_ref[...] == riota) & inr
    sel1 = (d1_ref[...] == riota) & inr
    gb = (jnp.where(sel0, rw0_ref[...], 0.0)
          + jnp.where(sel1, rw1_ref[...], 0.0))
    out_ref[...] += lax.dot_general(
        gb.astype(jnp.bfloat16), y.astype(jnp.bfloat16),
        (((1,), (0,)), ((), ())), preferred_element_type=jnp.float32)


def _ffn(meta, d0c, d1c, w0c, w1c, xs, w1, w1_up, w2):
    grid_spec = pltpu.PrefetchScalarGridSpec(
        num_scalar_prefetch=1,
        grid=(NT,),
        in_specs=[
            pl.BlockSpec((S, 1), lambda i, m: (0, 0)),
            pl.BlockSpec((S, 1), lambda i, m: (0, 0)),
            pl.BlockSpec((S, 1), lambda i, m: (0, 0)),
            pl.BlockSpec((S, 1), lambda i, m: (0, 0)),
            pl.BlockSpec((TM, H), lambda i, m: (m[i, 1], 0)),
            pl.BlockSpec((I, H), lambda i, m: (m[i, 0], 0)),
            pl.BlockSpec((I, H), lambda i, m: (m[i, 0], 0)),
            pl.BlockSpec((H, I), lambda i, m: (m[i, 0], 0)),
        ],
        out_specs=pl.BlockSpec((S, H), lambda i, m: (0, 0)),
    )
    return pl.pallas_call(
        _ffn_body,
        grid_spec=grid_spec,
        out_shape=jax.ShapeDtypeStruct((S, H), jnp.float32),
    )(meta, d0c, d1c, w0c, w1c, xs, w1, w1_up, w2)


def _dispatch_body(x_hbm, d0_hbm, d1_hbm, xs_hbm, rows_v, d0_v, d1_v,
                   sem0, sem1):
    wid = lax.axis_index("s") * NC + lax.axis_index("c")
    base = wid * TOK_W
    pltpu.sync_copy(x_hbm.at[pl.ds(base, TOK_W)], rows_v)
    pltpu.sync_copy(d0_hbm.at[pl.ds(base, TOK_W)], d0_v)
    pltpu.sync_copy(d1_hbm.at[pl.ds(base, TOK_W)], d1_v)
    c0 = pltpu.async_copy(rows_v, xs_hbm.at[d0_v], sem0)
    c1 = pltpu.async_copy(rows_v, xs_hbm.at[d1_v], sem1)
    c0.wait()
    c1.wait()


def _dispatch(x2, d0, d1):
    mesh = plsc.VectorSubcoreMesh(core_axis_name="c", subcore_axis_name="s",
                                  num_cores=NC, num_subcores=NS)
    return pl.kernel(
        _dispatch_body,
        out_type=jax.ShapeDtypeStruct((P, H), jnp.float32),
        mesh=mesh,
        scratch_types=[
            pltpu.VMEM((TOK_W, H), jnp.float32),
            pltpu.VMEM((TOK_W,), jnp.int32),
            pltpu.VMEM((TOK_W,), jnp.int32),
            pltpu.SemaphoreType.DMA,
            pltpu.SemaphoreType.DMA,
        ],
    )(x2, d0, d1)


def kernel(x, gate_w, w1, w1_up, w2):
    orig_shape = x.shape
    x2 = x.reshape(S, H)
    d0c, d1c, w0c, w1c, meta = _router_plan(x2, gate_w)
    d0 = d0c.reshape(S)
    d1 = d1c.reshape(S)
    xs = _dispatch(x2, d0, d1)
    out = _ffn(meta, d0c, d1c, w0c, w1c, xs, w1, w1_up, w2)
    return out.reshape(orig_shape)


# speedup vs baseline: 1.0626x; 1.0626x over previous
import functools

import jax
import jax.numpy as jnp
from jax import lax
from jax.experimental import pallas as pl
from jax.experimental.pallas import tpu as pltpu
from jax.experimental.pallas import tpu_sc as plsc

H = 1024
I = 704
E = 64
S = 2048
P = S * 2
TM = 256
NT = P // TM + E

NC = 2
NS = 16
NW = NC * NS
TOK_W = S // NW
CH = 16


def _router_plan_body(x_ref, gw_ref, d0_ref, d1_ref, w0_ref, w1_ref, meta_ref):
    x = x_ref[...]
    gw = gw_ref[...]
    logits = lax.dot_general(
        x.astype(jnp.bfloat16), gw.astype(jnp.bfloat16),
        (((1,), (1,)), ((), ())),
        preferred_element_type=jnp.float32)

    col = lax.broadcasted_iota(jnp.int32, (S, E), 1)
    m1 = jnp.max(logits, axis=1, keepdims=True)
    a1 = jnp.min(jnp.where(logits == m1, col, E), axis=1, keepdims=True)
    logits2 = jnp.where(col == a1, -jnp.inf, logits)
    m2 = jnp.max(logits2, axis=1, keepdims=True)
    a2 = jnp.min(jnp.where(logits2 == m2, col, E), axis=1, keepdims=True)

    t = jnp.exp(m2 - m1)
    w0_ref[...] = 1.0 / (1.0 + t)
    w1_ref[...] = t / (1.0 + t)

    oh0 = (col == a1).astype(jnp.float32)
    oh1 = (col == a2).astype(jnp.float32)
    csum = oh0 + oh1
    k = 1
    while k < S:
        pad = jnp.zeros((k, E), jnp.float32)
        csum = csum + jnp.concatenate([pad, csum[:-k, :]], axis=0)
        k *= 2
    cnt = csum[S - 1:S, :]

    er = lax.broadcasted_iota(jnp.int32, (E, E), 0)
    ec = lax.broadcasted_iota(jnp.int32, (E, E), 1)
    off = jnp.sum(jnp.where(ec < er, jnp.broadcast_to(cnt, (E, E)), 0.0),
                  axis=1).reshape(1, E)

    rank0 = jnp.sum(oh0 * csum, axis=1, keepdims=True) - 1.0
    rank1 = jnp.sum(oh1 * csum, axis=1, keepdims=True) - 1.0
    offb = jnp.broadcast_to(off, (S, E))
    d0 = jnp.sum(oh0 * offb, axis=1, keepdims=True) + rank0
    d1 = jnp.sum(oh1 * offb, axis=1, keepdims=True) + rank1
    d0_ref[...] = d0.astype(jnp.int32)
    d1_ref[...] = d1.astype(jnp.int32)

    tlo = jnp.floor(off / TM)
    thi = jnp.floor((off + cnt - 1.0) / TM)
    ntile = jnp.where(cnt > 0.0, thi - tlo + 1.0, 0.0)
    cinc = jnp.sum(jnp.where(ec <= er, jnp.broadcast_to(ntile, (E, E)), 0.0),
                   axis=1).reshape(1, E)
    cexc = cinc - ntile
    total = jnp.sum(ntile)

    s_idx = lax.broadcasted_iota(jnp.int32, (NT, 1), 0).astype(jnp.float32)
    s_c = jnp.minimum(s_idx, total - 1.0)
    cincb = jnp.broadcast_to(cinc, (NT, E))
    e_of_s = jnp.sum((s_c >= cincb).astype(jnp.float32), axis=1,
                     keepdims=True)
    colN = lax.broadcasted_iota(jnp.int32, (NT, E), 1).astype(jnp.float32)
    ohs = (colN == e_of_s).astype(jnp.float32)
    cexc_s = jnp.sum(ohs * jnp.broadcast_to(cexc, (NT, E)), axis=1,
                     keepdims=True)
    tlo_s = jnp.sum(ohs * jnp.broadcast_to(tlo, (NT, E)), axis=1,
                    keepdims=True)
    off_s = jnp.sum(ohs * jnp.broadcast_to(off, (NT, E)), axis=1,
                    keepdims=True)
    end_s = jnp.sum(ohs * jnp.broadcast_to(off + cnt, (NT, E)), axis=1,
                    keepdims=True)
    tile_s = tlo_s + (s_c - cexc_s)
    row_lo = jnp.maximum(off_s, tile_s * TM)
    row_hi = jnp.minimum(end_s, (tile_s + 1.0) * TM)
    valid = s_idx < total
    row_lo = jnp.where(valid, row_lo, 0.0)
    row_hi = jnp.where(valid, row_hi, 0.0)
    meta = jnp.concatenate([e_of_s, tile_s, row_lo, row_hi], axis=1)
    meta_ref[...] = meta.astype(jnp.int32)


def _router_plan(x2, gate_w):
    return pl.pallas_call(
        _router_plan_body,
        out_shape=(
            jax.ShapeDtypeStruct((S, 1), jnp.int32),
            jax.ShapeDtypeStruct((S, 1), jnp.int32),
            jax.ShapeDtypeStruct((S, 1), jnp.float32),
            jax.ShapeDtypeStruct((S, 1), jnp.float32),
            jax.ShapeDtypeStruct((NT, 4), jnp.int32),
        ),
    )(x2, gate_w)


def _ffn_body(meta_ref, d0_ref, d1_ref, rw0_ref, rw1_ref, xs_ref,
              w1_ref, up_ref, w2_ref, out_ref, ys_scr):
    i = pl.program_id(0)
    row_lo = meta_ref[i, 2]
    row_hi = meta_ref[i, 3]
    base = meta_ref[i, 1] * TM

    xb = xs_ref[...].astype(jnp.bfloat16)
    w1b = w1_ref[...].astype(jnp.bfloat16)
    upb = up_ref[...].astype(jnp.bfloat16)
    w2b = w2_ref[...].astype(jnp.bfloat16)
    nt = (((1,), (1,)), ((), ()))
    g = lax.dot_general(xb, w1b, nt, preferred_element_type=jnp.float32)
    u = lax.dot_general(xb, upb, nt, preferred_element_type=jnp.float32)
    h = (g / (1.0 + jnp.exp(-g))) * u
    y = lax.dot_general(h.astype(jnp.bfloat16), w2b, nt,
                        preferred_element_type=jnp.float32)

    rows = base + lax.broadcasted_iota(jnp.int32, (TM, 1), 0)
    rmask = (rows >= row_lo) & (rows < row_hi)
    sl = pl.ds(base, TM)
    ys_scr[sl, :] = jnp.where(rmask, y.astype(jnp.bfloat16), ys_scr[sl, :])

    @pl.when(i == NT - 1)
    def _combine():
        d0 = d0_ref[...]
        d1 = d1_ref[...]
        rw0 = rw0_ref[...]
        rw1 = rw1_ref[...]
        out_ref[...] = jnp.zeros((S, H), jnp.float32)
        for qb in range(P // H):
            qiota = qb * H + lax.broadcasted_iota(jnp.int32, (S, H), 1)
            gq = (jnp.where(d0 == qiota, rw0, 0.0)
                  + jnp.where(d1 == qiota, rw1, 0.0))
            out_ref[...] += lax.dot_general(
                gq.astype(jnp.bfloat16), ys_scr[pl.ds(qb * H, H), :],
                (((1,), (0,)), ((), ())),
                preferred_element_type=jnp.float32)


def _ffn(meta, d0c, d1c, w0c, w1c, xs, w1, w1_up, w2):
    grid_spec = pltpu.PrefetchScalarGridSpec(
        num_scalar_prefetch=1,
        grid=(NT,),
        in_specs=[
            pl.BlockSpec((S, 1), lambda i, m: (0, 0)),
            pl.BlockSpec((S, 1), lambda i, m: (0, 0)),
            pl.BlockSpec((S, 1), lambda i, m: (0, 0)),
            pl.BlockSpec((S, 1), lambda i, m: (0, 0)),
            pl.BlockSpec((TM, H), lambda i, m: (m[i, 1], 0)),
            pl.BlockSpec((I, H), lambda i, m: (m[i, 0], 0)),
            pl.BlockSpec((I, H), lambda i, m: (m[i, 0], 0)),
            pl.BlockSpec((H, I), lambda i, m: (m[i, 0], 0)),
        ],
        out_specs=pl.BlockSpec((S, H), lambda i, m: (0, 0)),
        scratch_shapes=[pltpu.VMEM((P, H), jnp.bfloat16)],
    )
    return pl.pallas_call(
        _ffn_body,
        grid_spec=grid_spec,
        out_shape=jax.ShapeDtypeStruct((S, H), jnp.float32),
    )(meta, d0c, d1c, w0c, w1c, xs, w1, w1_up, w2)


def _dispatch_body(x_hbm, d0_hbm, d1_hbm, xs_hbm, rows_v, d0_v, d1_v,
                   sem0, sem1):
    wid = lax.axis_index("s") * NC + lax.axis_index("c")
    base = wid * TOK_W
    pltpu.sync_copy(x_hbm.at[pl.ds(base, TOK_W)], rows_v)
    pltpu.sync_copy(d0_hbm.at[pl.ds(base, TOK_W)], d0_v)
    pltpu.sync_copy(d1_hbm.at[pl.ds(base, TOK_W)], d1_v)
    c0 = pltpu.async_copy(rows_v, xs_hbm.at[d0_v], sem0)
    c1 = pltpu.async_copy(rows_v, xs_hbm.at[d1_v], sem1)
    c0.wait()
    c1.wait()


def _dispatch(x2, d0, d1):
    mesh = plsc.VectorSubcoreMesh(core_axis_name="c", subcore_axis_name="s",
                                  num_cores=NC, num_subcores=NS)
    return pl.kernel(
        _dispatch_body,
        out_type=jax.ShapeDtypeStruct((P, H), jnp.float32),
        mesh=mesh,
        scratch_types=[
            pltpu.VMEM((TOK_W, H), jnp.float32),
            pltpu.VMEM((TOK_W,), jnp.int32),
            pltpu.VMEM((TOK_W,), jnp.int32),
            pltpu.SemaphoreType.DMA,
            pltpu.SemaphoreType.DMA,
        ],
    )(x2, d0, d1)


def kernel(x, gate_w, w1, w1_up, w2):
    orig_shape = x.shape
    x2 = x.reshape(S, H)
    d0c, d1c, w0c, w1c, meta = _router_plan(x2, gate_w)
    d0 = d0c.reshape(S)
    d1 = d1c.reshape(S)
    xs = _dispatch(x2, d0, d1)
    out = _ffn(meta, d0c, d1c, w0c, w1c, xs, w1, w1_up, w2)
    return out.reshape(orig_shape)
